# baseline (speedup 1.0000x reference)
import jax
import jax.numpy as jnp
from jax.experimental import pallas as pl

TOPK = 64
R = 16


def _kern(s_ref, w1t_ref, b1_ref, w2t_ref, b2_ref, w3t_ref, b3_ref, w4_ref,
          b4_ref, out_ref, mask_ref, idx_ref):
    i = pl.program_id(0)
    n = s_ref.shape[2]
    x0 = s_ref[0]
    x1 = s_ref[1]
    dn2 = jnp.sqrt(x0 * x0 + x1 * x1 + 1e-6)
    iota = jax.lax.broadcasted_iota(jnp.int32, (R, n), 1).astype(jnp.float32)

    vals = dn2
    idx_cols = []
    for _ in range(TOPK):
        m = jnp.min(vals, axis=1, keepdims=True)
        cand = jnp.where(vals == m, iota, jnp.float32(n))
        sel = jnp.min(cand, axis=1, keepdims=True)
        idx_cols.append(sel)
        vals = jnp.where(iota == sel, jnp.float32(jnp.inf), vals)
    idxf = jnp.concatenate(idx_cols, axis=1)
    idx_ref[...] = idxf.astype(jnp.int32)

    iota_col = jax.lax.broadcasted_iota(jnp.int32, (n, TOPK), 0).astype(jnp.float32)
    base = (i * R).astype(jnp.float32)
    xt_chunks = []
    mask_chunks = []
    for p in range(R):
        row_idx = idxf[p][None, :]
        onehot = (iota_col == row_idx).astype(jnp.float32)
        g = jnp.dot(s_ref[:, p, :], onehot,
                    preferred_element_type=jnp.float32)
        g0 = g[0:1]
        g1 = g[1:2]
        dist = jnp.sqrt(g0 * g0 + g1 * g1 + 1e-4)
        eye = (row_idx == base + p).astype(jnp.float32)
        margin = dist - g[4:5]
        xt_chunks.append(jnp.concatenate([g[0:4], eye, margin], axis=0))
        mask_chunks.append((dist <= 1.0).astype(jnp.float32))
    xt = jnp.concatenate(xt_chunks, axis=1)
    maskf = jnp.concatenate(mask_chunks, axis=1)

    h = jnp.maximum(jnp.dot(w1t_ref[...], xt,
                            preferred_element_type=jnp.float32) + b1_ref[...], 0.0)
    h = jnp.maximum(jnp.dot(w2t_ref[...], h,
                            preferred_element_type=jnp.float32) + b2_ref[...], 0.0)
    h = jnp.maximum(jnp.dot(w3t_ref[...], h,
                            preferred_element_type=jnp.float32) + b3_ref[...], 0.0)
    o = jnp.sum(h * w4_ref[...], axis=0, keepdims=True) + b4_ref[...]
    o = o * maskf
    out_ref[...] = o[None]
    mask_ref[...] = maskf[None]


def kernel(x, r, W1, b1, W2, b2, W3, b3, W4, b4):
    n = x.shape[0]
    f32 = jnp.float32
    s = jnp.concatenate([jnp.transpose(x, (2, 0, 1)), r[None]], axis=0)
    w1t = W1.T
    w2t = W2.T
    w3t = W3.T
    b1c = b1[:, None]
    b2c = b2[:, None]
    b3c = b3[:, None]
    b4c = b4[:, None]

    grid = n // R
    full = lambda a: pl.BlockSpec(a.shape, lambda i: (0,) * a.ndim)
    out, mask, idx = pl.pallas_call(
        _kern,
        grid=(grid,),
        in_specs=[
            pl.BlockSpec((5, R, n), lambda i: (0, i, 0)),
            full(w1t), full(b1c), full(w2t), full(b2c),
            full(w3t), full(b3c), full(W4), full(b4c),
        ],
        out_specs=[
            pl.BlockSpec((1, 1, R * TOPK), lambda i: (i, 0, 0)),
            pl.BlockSpec((1, 1, R * TOPK), lambda i: (i, 0, 0)),
            pl.BlockSpec((R, TOPK), lambda i: (i, 0)),
        ],
        out_shape=[
            jax.ShapeDtypeStruct((grid, 1, R * TOPK), f32),
            jax.ShapeDtypeStruct((grid, 1, R * TOPK), f32),
            jax.ShapeDtypeStruct((n, TOPK), jnp.int32),
        ],
    )(s, w1t, b1c, w2t, b2c, w3t, b3c, W4, b4c)
    return (out.reshape(n, TOPK, 1), mask.reshape(n, TOPK, 1), idx)


# speedup vs baseline: 1.5628x; 1.5628x over previous
import jax
import jax.numpy as jnp
from jax.experimental import pallas as pl

TOPK = 64
R = 32


def _kern(s_ref, w1t_ref, b1_ref, w2t_ref, b2_ref, w3t_ref, b3_ref, w4_ref,
          b4_ref, out_ref, mask_ref, idx_ref):
    i = pl.program_id(0)
    n = s_ref.shape[2]
    x0 = s_ref[0]
    x1 = s_ref[1]
    dn2 = jnp.sqrt(x0 * x0 + x1 * x1 + 1e-6)
    iota = jax.lax.broadcasted_iota(jnp.int32, (R, n), 1).astype(jnp.float32)

    vals = dn2
    idx_cols = []
    for _ in range(TOPK):
        m = jnp.min(vals, axis=1, keepdims=True)
        cand = jnp.where(vals == m, iota, jnp.float32(n))
        sel = jnp.min(cand, axis=1, keepdims=True)
        idx_cols.append(sel)
        vals = jnp.where(iota == sel, jnp.float32(jnp.inf), vals)
    idxf = jnp.concatenate(idx_cols, axis=1)
    idx_ref[...] = idxf.astype(jnp.int32)

    iota_col = jax.lax.broadcasted_iota(jnp.int32, (n, TOPK), 0).astype(jnp.float32)
    base = (i * R).astype(jnp.float32)
    xt_chunks = []
    mask_chunks = []
    for p in range(R):
        row_idx = idxf[p][None, :]
        onehot = (iota_col == row_idx).astype(jnp.float32)
        g = jnp.dot(s_ref[:, p, :], onehot,
                    preferred_element_type=jnp.float32)
        g0 = g[0:1]
        g1 = g[1:2]
        dist = jnp.sqrt(g0 * g0 + g1 * g1 + 1e-4)
        eye = (row_idx == base + p).astype(jnp.float32)
        margin = dist - g[4:5]
        xt_chunks.append(jnp.concatenate([g[0:4], eye, margin], axis=0))
        mask_chunks.append((dist <= 1.0).astype(jnp.float32))
    xt = jnp.concatenate(xt_chunks, axis=1)
    maskf = jnp.concatenate(mask_chunks, axis=1)

    h = jnp.maximum(jnp.dot(w1t_ref[...], xt,
                            preferred_element_type=jnp.float32) + b1_ref[...], 0.0)
    h = jnp.maximum(jnp.dot(w2t_ref[...], h,
                            preferred_element_type=jnp.float32) + b2_ref[...], 0.0)
    h = jnp.maximum(jnp.dot(w3t_ref[...], h,
                            preferred_element_type=jnp.float32) + b3_ref[...], 0.0)
    o = jnp.sum(h * w4_ref[...], axis=0, keepdims=True) + b4_ref[...]
    o = o * maskf
    out_ref[...] = o[None]
    mask_ref[...] = maskf[None]


def kernel(x, r, W1, b1, W2, b2, W3, b3, W4, b4):
    n = x.shape[0]
    f32 = jnp.float32
    s = jnp.concatenate([jnp.transpose(x, (2, 0, 1)), r[None]], axis=0)
    w1t = W1.T
    w2t = W2.T
    w3t = W3.T
    b1c = b1[:, None]
    b2c = b2[:, None]
    b3c = b3[:, None]
    b4c = b4[:, None]

    grid = n // R
    full = lambda a: pl.BlockSpec(a.shape, lambda i: (0,) * a.ndim)
    out, mask, idx = pl.pallas_call(
        _kern,
        grid=(grid,),
        in_specs=[
            pl.BlockSpec((5, R, n), lambda i: (0, i, 0)),
            full(w1t), full(b1c), full(w2t), full(b2c),
            full(w3t), full(b3c), full(W4), full(b4c),
        ],
        out_specs=[
            pl.BlockSpec((1, 1, R * TOPK), lambda i: (i, 0, 0)),
            pl.BlockSpec((1, 1, R * TOPK), lambda i: (i, 0, 0)),
            pl.BlockSpec((R, TOPK), lambda i: (i, 0)),
        ],
        out_shape=[
            jax.ShapeDtypeStruct((grid, 1, R * TOPK), f32),
            jax.ShapeDtypeStruct((grid, 1, R * TOPK), f32),
            jax.ShapeDtypeStruct((n, TOPK), jnp.int32),
        ],
    )(s, w1t, b1c, w2t, b2c, w3t, b3c, W4, b4c)
    return (out.reshape(n, TOPK, 1), mask.reshape(n, TOPK, 1), idx)


# speedup vs baseline: 2.2824x; 1.4604x over previous
import jax
import jax.numpy as jnp
from jax.experimental import pallas as pl

TOPK = 64
R = 64


def _kern(s_ref, w1t_ref, b1_ref, w2t_ref, b2_ref, w3t_ref, b3_ref, w4_ref,
          b4_ref, out_ref, mask_ref, idx_ref):
    i = pl.program_id(0)
    n = s_ref.shape[2]
    x0 = s_ref[0]
    x1 = s_ref[1]
    dn2 = jnp.sqrt(x0 * x0 + x1 * x1 + 1e-6)
    iota = jax.lax.broadcasted_iota(jnp.int32, (R, n), 1).astype(jnp.float32)

    vals = dn2
    idx_cols = []
    for _ in range(TOPK):
        m = jnp.min(vals, axis=1, keepdims=True)
        cand = jnp.where(vals == m, iota, jnp.float32(n))
        sel = jnp.min(cand, axis=1, keepdims=True)
        idx_cols.append(sel)
        vals = jnp.where(iota == sel, jnp.float32(jnp.inf), vals)
    idxf = jnp.concatenate(idx_cols, axis=1)
    idx_ref[...] = idxf.astype(jnp.int32)

    iota_col = jax.lax.broadcasted_iota(jnp.int32, (n, TOPK), 0).astype(jnp.float32)
    base = (i * R).astype(jnp.float32)
    xt_chunks = []
    mask_chunks = []
    for p in range(R):
        row_idx = idxf[p][None, :]
        onehot = (iota_col == row_idx).astype(jnp.float32)
        g = jnp.dot(s_ref[:, p, :], onehot,
                    preferred_element_type=jnp.float32)
        g0 = g[0:1]
        g1 = g[1:2]
        dist = jnp.sqrt(g0 * g0 + g1 * g1 + 1e-4)
        eye = (row_idx == base + p).astype(jnp.float32)
        margin = dist - g[4:5]
        xt_chunks.append(jnp.concatenate([g[0:4], eye, margin], axis=0))
        mask_chunks.append((dist <= 1.0).astype(jnp.float32))
    xt = jnp.concatenate(xt_chunks, axis=1)
    maskf = jnp.concatenate(mask_chunks, axis=1)

    h = jnp.maximum(jnp.dot(w1t_ref[...], xt,
                            preferred_element_type=jnp.float32) + b1_ref[...], 0.0)
    h = jnp.maximum(jnp.dot(w2t_ref[...], h,
                            preferred_element_type=jnp.float32) + b2_ref[...], 0.0)
    h = jnp.maximum(jnp.dot(w3t_ref[...], h,
                            preferred_element_type=jnp.float32) + b3_ref[...], 0.0)
    o = jnp.sum(h * w4_ref[...], axis=0, keepdims=True) + b4_ref[...]
    o = o * maskf
    out_ref[...] = o[None]
    mask_ref[...] = maskf[None]


def kernel(x, r, W1, b1, W2, b2, W3, b3, W4, b4):
    n = x.shape[0]
    f32 = jnp.float32
    s = jnp.concatenate([jnp.transpose(x, (2, 0, 1)), r[None]], axis=0)
    w1t = W1.T
    w2t = W2.T
    w3t = W3.T
    b1c = b1[:, None]
    b2c = b2[:, None]
    b3c = b3[:, None]
    b4c = b4[:, None]

    grid = n // R
    full = lambda a: pl.BlockSpec(a.shape, lambda i: (0,) * a.ndim)
    out, mask, idx = pl.pallas_call(
        _kern,
        grid=(grid,),
        in_specs=[
            pl.BlockSpec((5, R, n), lambda i: (0, i, 0)),
            full(w1t), full(b1c), full(w2t), full(b2c),
            full(w3t), full(b3c), full(W4), full(b4c),
        ],
        out_specs=[
            pl.BlockSpec((1, 1, R * TOPK), lambda i: (i, 0, 0)),
            pl.BlockSpec((1, 1, R * TOPK), lambda i: (i, 0, 0)),
            pl.BlockSpec((R, TOPK), lambda i: (i, 0)),
        ],
        out_shape=[
            jax.ShapeDtypeStruct((grid, 1, R * TOPK), f32),
            jax.ShapeDtypeStruct((grid, 1, R * TOPK), f32),
            jax.ShapeDtypeStruct((n, TOPK), jnp.int32),
        ],
    )(s, w1t, b1c, w2t, b2c, w3t, b3c, W4, b4c)
    return (out.reshape(n, TOPK, 1), mask.reshape(n, TOPK, 1), idx)


# speedup vs baseline: 2.8094x; 1.2309x over previous
import jax
import jax.numpy as jnp
from jax.experimental import pallas as pl

TOPK = 64
R = 128


def _kern(s_ref, w1t_ref, b1_ref, w2t_ref, b2_ref, w3t_ref, b3_ref, w4_ref,
          b4_ref, out_ref, mask_ref, idx_ref):
    i = pl.program_id(0)
    n = s_ref.shape[2]
    x0 = s_ref[0]
    x1 = s_ref[1]
    dn2 = jnp.sqrt(x0 * x0 + x1 * x1 + 1e-6)
    iota = jax.lax.broadcasted_iota(jnp.int32, (R, n), 1).astype(jnp.float32)

    vals = dn2
    idx_cols = []
    for _ in range(TOPK):
        m = jnp.min(vals, axis=1, keepdims=True)
        cand = jnp.where(vals == m, iota, jnp.float32(n))
        sel = jnp.min(cand, axis=1, keepdims=True)
        idx_cols.append(sel)
        vals = jnp.where(iota == sel, jnp.float32(jnp.inf), vals)
    idxf = jnp.concatenate(idx_cols, axis=1)
    idx_ref[...] = idxf.astype(jnp.int32)

    iota_col = jax.lax.broadcasted_iota(jnp.int32, (n, TOPK), 0).astype(jnp.float32)
    base = (i * R).astype(jnp.float32)
    xt_chunks = []
    mask_chunks = []
    for p in range(R):
        row_idx = idxf[p][None, :]
        onehot = (iota_col == row_idx).astype(jnp.float32)
        g = jnp.dot(s_ref[:, p, :], onehot,
                    preferred_element_type=jnp.float32)
        g0 = g[0:1]
        g1 = g[1:2]
        dist = jnp.sqrt(g0 * g0 + g1 * g1 + 1e-4)
        eye = (row_idx == base + p).astype(jnp.float32)
        margin = dist - g[4:5]
        xt_chunks.append(jnp.concatenate([g[0:4], eye, margin], axis=0))
        mask_chunks.append((dist <= 1.0).astype(jnp.float32))
    xt = jnp.concatenate(xt_chunks, axis=1)
    maskf = jnp.concatenate(mask_chunks, axis=1)

    h = jnp.maximum(jnp.dot(w1t_ref[...], xt,
                            preferred_element_type=jnp.float32) + b1_ref[...], 0.0)
    h = jnp.maximum(jnp.dot(w2t_ref[...], h,
                            preferred_element_type=jnp.float32) + b2_ref[...], 0.0)
    h = jnp.maximum(jnp.dot(w3t_ref[...], h,
                            preferred_element_type=jnp.float32) + b3_ref[...], 0.0)
    o = jnp.sum(h * w4_ref[...], axis=0, keepdims=True) + b4_ref[...]
    o = o * maskf
    out_ref[...] = o[None]
    mask_ref[...] = maskf[None]


def kernel(x, r, W1, b1, W2, b2, W3, b3, W4, b4):
    n = x.shape[0]
    f32 = jnp.float32
    s = jnp.concatenate([jnp.transpose(x, (2, 0, 1)), r[None]], axis=0)
    w1t = W1.T
    w2t = W2.T
    w3t = W3.T
    b1c = b1[:, None]
    b2c = b2[:, None]
    b3c = b3[:, None]
    b4c = b4[:, None]

    grid = n // R
    full = lambda a: pl.BlockSpec(a.shape, lambda i: (0,) * a.ndim)
    out, mask, idx = pl.pallas_call(
        _kern,
        grid=(grid,),
        in_specs=[
            pl.BlockSpec((5, R, n), lambda i: (0, i, 0)),
            full(w1t), full(b1c), full(w2t), full(b2c),
            full(w3t), full(b3c), full(W4), full(b4c),
        ],
        out_specs=[
            pl.BlockSpec((1, 1, R * TOPK), lambda i: (i, 0, 0)),
            pl.BlockSpec((1, 1, R * TOPK), lambda i: (i, 0, 0)),
            pl.BlockSpec((R, TOPK), lambda i: (i, 0)),
        ],
        out_shape=[
            jax.ShapeDtypeStruct((grid, 1, R * TOPK), f32),
            jax.ShapeDtypeStruct((grid, 1, R * TOPK), f32),
            jax.ShapeDtypeStruct((n, TOPK), jnp.int32),
        ],
    )(s, w1t, b1c, w2t, b2c, w3t, b3c, W4, b4c)
    return (out.reshape(n, TOPK, 1), mask.reshape(n, TOPK, 1), idx)
